# baseline (speedup 1.0000x reference)
import dataclasses
import functools

import jax
import jax.numpy as jnp
from jax.experimental import pallas as pl
from jax.experimental.pallas import tpu as pltpu
from jax.experimental.pallas import tpu_sc as plsc


_SC_CORES = 2
_SC_SUBCORES = 16
_LANES = 16
_CHUNK = 128


def _gather_dims_sc(table_wide, idx, dim, num_cols):
    batch = idx.shape[0]
    num_workers = _SC_CORES * _SC_SUBCORES
    d_per_w = dim // num_workers
    n_chunks = batch // _CHUNK
    n_units = d_per_w * n_chunks
    mesh = plsc.VectorSubcoreMesh(core_axis_name="c", subcore_axis_name="s")

    @functools.partial(
        pl.kernel, mesh=mesh,
        out_type=jax.ShapeDtypeStruct((dim, batch), jnp.float32),
        compiler_params=dataclasses.replace(
            pltpu.CompilerParams(), needs_layout_passes=False),
        scratch_types=[
            pltpu.VMEM((batch,), jnp.int32),
            pltpu.VMEM((2, _CHUNK), jnp.int32),
            pltpu.VMEM((2, _CHUNK), jnp.int32),
            pltpu.VMEM((2, _CHUNK, _CHUNK), jnp.float32),
            pltpu.VMEM((d_per_w, batch), jnp.float32),
            pltpu.SemaphoreType.DMA,
            pltpu.SemaphoreType.DMA,
        ],
    )
    def gather_kernel(table_hbm, idx_hbm, out_hbm,
                      idx_v, widx_v, lane_v, rows_v, sel_v, sem_a, sem_b):
        wid = jax.lax.axis_index("s") * _SC_CORES + jax.lax.axis_index("c")
        d_base = wid * d_per_w
        sems = (sem_a, sem_b)
        pltpu.sync_copy(idx_hbm, idx_v)

        def issue(unit, buf):
            k, c = divmod(unit, n_chunks)
            d_scaled = (d_base + k) * num_cols
            for t in range(_CHUNK // _LANES):
                u16 = idx_v[pl.ds(c * _CHUNK + t * _LANES, _LANES)]
                e16 = u16 + d_scaled
                widx_v[buf, pl.ds(t * _LANES, _LANES)] = jnp.right_shift(e16, 7)
                lane_v[buf, pl.ds(t * _LANES, _LANES)] = jnp.bitwise_and(e16, 127)
            return pltpu.async_copy(
                table_hbm.at[widx_v.at[buf]], rows_v.at[buf], sems[buf])

        handles = [None, None]
        handles[0] = issue(0, 0)
        for unit in range(n_units):
            buf = unit % 2
            if unit + 1 < n_units:
                handles[(unit + 1) % 2] = issue(unit + 1, (unit + 1) % 2)
            handles[buf].wait()
            k, c = divmod(unit, n_chunks)
            for t in range(_CHUNK // _LANES):
                r16 = jax.lax.broadcasted_iota(
                    jnp.int32, (_LANES,), 0) + t * _LANES
                l16 = lane_v[buf, pl.ds(t * _LANES, _LANES)]
                v16 = plsc.load_gather(rows_v.at[buf], [r16, l16])
                sel_v[k, pl.ds(c * _CHUNK + t * _LANES, _LANES)] = v16
            if c == n_chunks - 1:
                pltpu.sync_copy(sel_v.at[k], out_hbm.at[d_base + k])

    return gather_kernel(table_wide, idx)


_ITEM_BLOCK = 4096


def _scores_t_tc(users_emb_t, item_t):
    dim, batch = users_emb_t.shape
    num_items = item_t.shape[1]

    def score_kernel(u_ref, it_ref, o_ref):
        scores = jax.lax.dot_general(
            it_ref[...], u_ref[...],
            dimension_numbers=(((0,), (0,)), ((), ())),
            preferred_element_type=jnp.float32)
        o_ref[...] = 0.5 + 0.5 * jnp.tanh(0.5 * scores)

    return pl.pallas_call(
        score_kernel,
        grid=(pl.cdiv(num_items, _ITEM_BLOCK),),
        in_specs=[
            pl.BlockSpec((dim, batch), lambda i: (0, 0)),
            pl.BlockSpec((dim, _ITEM_BLOCK), lambda i: (0, i)),
        ],
        out_specs=pl.BlockSpec((_ITEM_BLOCK, batch), lambda i: (i, 0)),
        out_shape=jax.ShapeDtypeStruct((num_items, batch), jnp.float32),
    )(users_emb_t, item_t)


def kernel(users, embedding_user, embedding_item):
    num_users, dim = embedding_user.shape
    user_wide = embedding_user.T.reshape(num_users * dim // 128, 128)
    item_t = embedding_item.T
    users_emb_t = _gather_dims_sc(
        user_wide, users.astype(jnp.int32), dim, num_users)
    scores_t = _scores_t_tc(users_emb_t, item_t)
    return scores_t.T


# speedup vs baseline: 1.0833x; 1.0833x over previous
import dataclasses
import functools

import jax
import jax.numpy as jnp
from jax.experimental import pallas as pl
from jax.experimental.pallas import tpu as pltpu
from jax.experimental.pallas import tpu_sc as plsc


_SC_CORES = 2
_SC_SUBCORES = 16
_LANES = 16
_CHUNK = 128


def _widen_table_tc(table_t):
    dim, n = table_t.shape
    n_c = pl.cdiv(n, 128)
    c_per_step = 46
    steps = n_c // c_per_step

    def body(in_ref, o_ref):
        for cl in range(c_per_step):
            o_ref[pl.ds(cl * dim, dim), :] = in_ref[:, pl.ds(cl * 128, 128)]

    return pl.pallas_call(
        body,
        grid=(steps,),
        in_specs=[pl.BlockSpec((dim, c_per_step * 128), lambda i: (0, i))],
        out_specs=pl.BlockSpec((c_per_step * dim, 128), lambda i: (i, 0)),
        out_shape=jax.ShapeDtypeStruct((n_c * dim, 128), jnp.float32),
    )(table_t)


def _gather_dims_sc(table_wide, idx, dim, num_cols):
    batch = idx.shape[0]
    num_workers = _SC_CORES * _SC_SUBCORES
    d_per_w = dim // num_workers
    n_chunks = batch // _CHUNK
    n_units = d_per_w * n_chunks
    mesh = plsc.VectorSubcoreMesh(core_axis_name="c", subcore_axis_name="s")

    @functools.partial(
        pl.kernel, mesh=mesh,
        out_type=jax.ShapeDtypeStruct((dim, batch), jnp.float32),
        compiler_params=dataclasses.replace(
            pltpu.CompilerParams(), needs_layout_passes=False),
        scratch_types=[
            pltpu.VMEM((batch,), jnp.int32),
            pltpu.VMEM((2, _CHUNK), jnp.int32),
            pltpu.VMEM((2, _CHUNK), jnp.int32),
            pltpu.VMEM((2, _CHUNK, _CHUNK), jnp.float32),
            pltpu.VMEM((d_per_w, batch), jnp.float32),
            pltpu.SemaphoreType.DMA,
            pltpu.SemaphoreType.DMA,
        ],
    )
    def gather_kernel(table_hbm, idx_hbm, out_hbm,
                      idx_v, widx_v, lane_v, rows_v, sel_v, sem_a, sem_b):
        wid = jax.lax.axis_index("s") * _SC_CORES + jax.lax.axis_index("c")
        d_base = wid * d_per_w
        sems = (sem_a, sem_b)
        pltpu.sync_copy(idx_hbm, idx_v)

        def issue(unit, buf):
            k, c = divmod(unit, n_chunks)
            d_cur = d_base + k
            for t in range(_CHUNK // _LANES):
                u16 = idx_v[pl.ds(c * _CHUNK + t * _LANES, _LANES)]
                widx_v[buf, pl.ds(t * _LANES, _LANES)] = (
                    jnp.right_shift(u16, 7) * dim + d_cur)
                lane_v[buf, pl.ds(t * _LANES, _LANES)] = jnp.bitwise_and(u16, 127)
            return pltpu.async_copy(
                table_hbm.at[widx_v.at[buf]], rows_v.at[buf], sems[buf])

        handles = [None, None]
        handles[0] = issue(0, 0)
        for unit in range(n_units):
            buf = unit % 2
            if unit + 1 < n_units:
                handles[(unit + 1) % 2] = issue(unit + 1, (unit + 1) % 2)
            handles[buf].wait()
            k, c = divmod(unit, n_chunks)
            for t in range(_CHUNK // _LANES):
                r16 = jax.lax.broadcasted_iota(
                    jnp.int32, (_LANES,), 0) + t * _LANES
                l16 = lane_v[buf, pl.ds(t * _LANES, _LANES)]
                v16 = plsc.load_gather(rows_v.at[buf], [r16, l16])
                sel_v[k, pl.ds(c * _CHUNK + t * _LANES, _LANES)] = v16
            if c == n_chunks - 1:
                pltpu.sync_copy(sel_v.at[k], out_hbm.at[d_base + k])

    return gather_kernel(table_wide, idx)


_ITEM_BLOCK = 4096


def _scores_t_tc(users_emb_t, item_t):
    dim, batch = users_emb_t.shape
    num_items = item_t.shape[1]

    def score_kernel(u_ref, it_ref, o_ref):
        scores = jax.lax.dot_general(
            it_ref[...], u_ref[...],
            dimension_numbers=(((0,), (0,)), ((), ())),
            preferred_element_type=jnp.float32)
        o_ref[...] = 0.5 + 0.5 * jnp.tanh(0.5 * scores)

    return pl.pallas_call(
        score_kernel,
        grid=(pl.cdiv(num_items, _ITEM_BLOCK),),
        in_specs=[
            pl.BlockSpec((dim, batch), lambda i: (0, 0)),
            pl.BlockSpec((dim, _ITEM_BLOCK), lambda i: (0, i)),
        ],
        out_specs=pl.BlockSpec((_ITEM_BLOCK, batch), lambda i: (i, 0)),
        out_shape=jax.ShapeDtypeStruct((num_items, batch), jnp.float32),
    )(users_emb_t, item_t)


def kernel(users, embedding_user, embedding_item):
    num_users, dim = embedding_user.shape
    user_t = embedding_user.T
    item_t = embedding_item.T
    user_wide = _widen_table_tc(user_t)
    users_emb_t = _gather_dims_sc(
        user_wide, users.astype(jnp.int32), dim, num_users)
    scores_t = _scores_t_tc(users_emb_t, item_t)
    return scores_t.T
